# baseline (speedup 1.0000x reference)
import functools

import jax
import jax.numpy as jnp
from jax import lax
from jax.experimental import pallas as pl
from jax.experimental.pallas import tpu as pltpu, tpu_sc as plsc

T, N, E, D, H = 3, 10000, 320000, 128, 128
SLOPE = (1.0 / 8.0 + 1.0 / 3.0) / 2.0

NC, NS = 2, 16
NW = NC * NS
NP = 10240
DEG_NP = 2 * NP
EPW = E // NW
CH = 80
NCH = EPW // CH

DEG_TILE = DEG_NP // NS

_sc_mesh = plsc.VectorSubcoreMesh(
    core_axis_name="c", subcore_axis_name="s", num_cores=NC, num_subcores=NS)


@functools.partial(
    pl.kernel,
    mesh=_sc_mesh,
    compiler_params=pltpu.CompilerParams(needs_layout_passes=False),
    out_type=jax.ShapeDtypeStruct((NC, DEG_NP), jnp.float32),
    scratch_types=[
        pltpu.VMEM((EPW,), jnp.int32),
        pltpu.VMEM((DEG_NP,), jnp.float32),
        pltpu.VMEM((NS, DEG_TILE), jnp.float32),
        pltpu.VMEM_SHARED((NS, DEG_NP), jnp.float32),
    ],
)
def _sc_degrees(src_hbm, dst_hbm, out_hbm, idx_v, hist_v, red_v, hists_sh):
    cid = lax.axis_index("c")
    sid = lax.axis_index("s")
    wid = cid * NS + sid

    def zbody(i, carry):
        hist_v[pl.ds(i * 16, 16)] = jnp.zeros((16,), jnp.float32)
        return carry

    lax.fori_loop(0, DEG_NP // 16, zbody, 0)

    def count(idx_hbm, offset):
        pltpu.sync_copy(idx_hbm.at[pl.ds(wid * EPW, EPW)], idx_v)

        def body(i, carry):
            idx16 = idx_v[pl.ds(i * 16, 16)] + offset
            cnt, last = plsc.scan_count(idx16)
            plsc.addupdate_scatter(hist_v, [idx16],
                                   cnt.astype(jnp.float32), mask=last)
            return carry

        lax.fori_loop(0, EPW // 16, body, 0)

    count(src_hbm, 0)
    count(dst_hbm, NP)
    pltpu.sync_copy(hist_v, hists_sh.at[sid])
    plsc.subcore_barrier()
    pltpu.sync_copy(hists_sh.at[:, pl.ds(sid * DEG_TILE, DEG_TILE)], red_v)

    def rbody(i, carry):
        s = red_v[0, pl.ds(i * 16, 16)]
        for k in range(1, NS):
            s = s + red_v[k, pl.ds(i * 16, 16)]
        hist_v[pl.ds(i * 16, 16)] = s
        return carry

    lax.fori_loop(0, DEG_TILE // 16, rbody, 0)
    pltpu.sync_copy(hist_v.at[pl.ds(0, DEG_TILE)],
                    out_hbm.at[cid, pl.ds(sid * DEG_TILE, DEG_TILE)])


@functools.partial(
    pl.kernel,
    mesh=_sc_mesh,
    out_type=jax.ShapeDtypeStruct((NC, NP, H), jnp.float32),
    scratch_types=(
        [pltpu.VMEM((CH,), jnp.int32) for _ in range(4)]
        + [pltpu.VMEM((CH,), jnp.int32) for _ in range(4)]
        + [pltpu.VMEM((CH, H), jnp.float32) for _ in range(4)]
        + [pltpu.VMEM_SHARED((NP, H), jnp.float32)]
        + [pltpu.SemaphoreType.DMA for _ in range(8)]
    ),
)
def _sc_aggregate(hw_hbm, src_hbm, dst_hbm, out_hbm,
                  si0, si1, si2, si3, di0, di1, di2, di3,
                  b0, b1, b2, b3, agg_sh, s0, s1, s2, s3,
                  t0, t1, t2, t3):
    cid = lax.axis_index("c")
    sid = lax.axis_index("s")
    wid = cid * NS + sid
    rows = NP // NS
    sidx = (si0, si1, si2, si3)
    didx = (di0, di1, di2, di3)
    bufs = (b0, b1, b2, b3)
    sems = (s0, s1, s2, s3)
    ssems = (t0, t1, t2, t3)

    def zb(i, carry):
        for k in range(H // 16):
            b0[i, pl.ds(k * 16, 16)] = jnp.zeros((16,), jnp.float32)
        return carry

    lax.fori_loop(0, CH, zb, 0)
    for r in range(rows // CH):
        pltpu.sync_copy(b0,
                        agg_sh.at[pl.ds(sid * rows + r * CH, CH), :])
    plsc.subcore_barrier()

    base = wid * EPW

    def gather(j, b):
        pltpu.sync_copy(src_hbm.at[pl.ds(base + j * CH, CH)], sidx[b])
        pltpu.async_copy(hw_hbm.at[sidx[b]], bufs[b], sems[b])

    def gather_wait(b):
        pltpu.make_async_copy(hw_hbm.at[sidx[b]], bufs[b], sems[b]).wait()

    def scatter(j, b):
        pltpu.sync_copy(dst_hbm.at[pl.ds(base + j * CH, CH)], didx[b])
        pltpu.async_copy(bufs[b], agg_sh.at[didx[b]], ssems[b], add=True)

    def scatter_wait(b):
        pltpu.make_async_copy(bufs[b], agg_sh.at[didx[b]], ssems[b]).wait()

    gather(0, 0)
    gather(1, 1)
    gather(2, 2)
    gather_wait(0)
    scatter(0, 0)
    gather(3, 3)
    gather_wait(1)
    scatter(1, 1)

    def body(t, carry):
        j0 = 2 + 4 * t
        for m in range(4):
            scatter_wait(m)
            gather(j0 + m + 2, m)
            bc = (2 + m) % 4
            gather_wait(bc)
            scatter(j0 + m, bc)
        return carry

    lax.fori_loop(0, (NCH - 5) // 4, body, 0)
    scatter_wait(0)
    gather(NCH - 1, 0)
    gather_wait(2)
    scatter(NCH - 3, 2)
    scatter_wait(1)
    gather_wait(3)
    scatter(NCH - 2, 3)
    scatter_wait(2)
    gather_wait(0)
    scatter(NCH - 1, 0)
    scatter_wait(3)
    scatter_wait(0)

    plsc.subcore_barrier()
    pltpu.sync_copy(agg_sh.at[pl.ds(sid * rows, rows), :],
                    out_hbm.at[cid, pl.ds(sid * rows, rows), :])


def _mm(a, b):
    return jnp.dot(a, b, preferred_element_type=jnp.float32)


def _mm_t(a, b):
    return lax.dot_general(a, b, (((1,), (1,)), ((), ())),
                           preferred_element_type=jnp.float32)


def _gru_step(W, uW, uU, ub, rW, rU, rb, hW, hU, hb):
    update = jax.nn.sigmoid(_mm(uW, W) + _mm(uU, W) + ub)
    reset = jax.nn.sigmoid(_mm(rW, W) + _mm(rU, W) + rb)
    h_cap = jnp.tanh(_mm(hW, W) + _mm(hU, reset * W) + hb)
    return (1.0 - update) * W + update * h_cap


def _tc_weights_body(gcn_W0, gcn_W1,
                     g0_uW, g0_uU, g0_ub, g0_rW, g0_rU, g0_rb,
                     g0_hW, g0_hU, g0_hb,
                     g1_uW, g1_uU, g1_ub, g1_rW, g1_rU, g1_rb,
                     g1_hW, g1_hU, g1_hb,
                     att_qw, att_qb, att_kw, att_kb, att_vw, att_vb,
                     att_gate, wa0_ref, wa1_ref):
    def evolve(W0, uW, uU, ub, rW, rU, rb, hW, hU, hb):
        W = W0[...]
        for _ in range(T):
            W = _gru_step(W, uW[...], uU[...], ub[...], rW[...], rU[...],
                          rb[...], hW[...], hU[...], hb[...])
        return W

    def attention(W):
        Q = _mm_t(W, att_qw[...]) + att_qb[...]
        K = _mm_t(W, att_kw[...]) + att_kb[...]
        V = _mm_t(W, att_vw[...]) + att_vb[...]
        scores = _mm_t(Q, K) / jnp.sqrt(jnp.float32(H))
        m = jnp.max(scores, axis=-1, keepdims=True)
        e = jnp.exp(scores - m)
        att = _mm(e / jnp.sum(e, axis=-1, keepdims=True), V)
        g = jax.nn.sigmoid(att_gate[...])
        return g * att + (1.0 - g) * W

    W0 = evolve(gcn_W0, g0_uW, g0_uU, g0_ub, g0_rW, g0_rU, g0_rb,
                g0_hW, g0_hU, g0_hb)
    W1 = evolve(gcn_W1, g1_uW, g1_uU, g1_ub, g1_rW, g1_rU, g1_rb,
                g1_hW, g1_hU, g1_hb)
    wa0_ref[...] = attention(W0)
    wa1_ref[...] = attention(W1)


def _tc_node_body(x2, degp, wa0, hw0_ref, s_out_ref, s_in_ref):
    deg_src = degp[0, :N, :] + degp[1, :N, :]
    deg_dst = degp[0, NP:NP + N, :] + degp[1, NP:NP + N, :]
    s_out = lax.rsqrt(jnp.maximum(deg_src, 1.0))
    s_in = lax.rsqrt(jnp.maximum(deg_dst, 1.0))
    s_out_ref[...] = s_out
    s_in_ref[...] = s_in
    hw0_ref[...] = _mm(x2[...] * s_out, wa0[...])


def _tc_layer_body(aggp, s_in, s_out, wa1, hw1_ref):
    agg = aggp[0, :N, :] + aggp[1, :N, :]
    rst = agg * s_in[...]
    feat = jnp.where(rst >= 0, rst, SLOPE * rst)
    hw1_ref[...] = _mm(feat * s_out[...], wa1[...])


def _tc_final_body(aggp, s_in, out_ref):
    agg = aggp[0, :N, :] + aggp[1, :N, :]
    rst = agg * s_in[...]
    out_ref[...] = jnp.where(rst >= 0, rst, SLOPE * rst)


_f32 = jnp.float32

_tc_weights = pl.pallas_call(
    _tc_weights_body,
    out_shape=(jax.ShapeDtypeStruct((H, H), _f32),
               jax.ShapeDtypeStruct((H, H), _f32)),
)

_tc_node = pl.pallas_call(
    _tc_node_body,
    out_shape=(jax.ShapeDtypeStruct((N, H), _f32),
               jax.ShapeDtypeStruct((N, 1), _f32),
               jax.ShapeDtypeStruct((N, 1), _f32)),
)

_tc_layer = pl.pallas_call(
    _tc_layer_body,
    out_shape=jax.ShapeDtypeStruct((N, H), _f32),
)

_tc_final = pl.pallas_call(
    _tc_final_body,
    out_shape=jax.ShapeDtypeStruct((N, H), _f32),
)


def kernel(x, edge_index, gcn_W0, gcn_W1,
           g0_uW, g0_uU, g0_ub, g0_rW, g0_rU, g0_rb, g0_hW, g0_hU, g0_hb,
           g1_uW, g1_uU, g1_ub, g1_rW, g1_rU, g1_rb, g1_hW, g1_hU, g1_hb,
           att_qw, att_qb, att_kw, att_kb, att_vw, att_vb, att_gate):
    x2 = x[T - 1]
    src = edge_index[T - 1, 0]
    dst = edge_index[T - 1, 1]

    degp = _sc_degrees(src, dst).reshape(NC, DEG_NP, 1)
    wa0, wa1 = _tc_weights(gcn_W0, gcn_W1,
                           g0_uW, g0_uU, g0_ub, g0_rW, g0_rU, g0_rb,
                           g0_hW, g0_hU, g0_hb,
                           g1_uW, g1_uU, g1_ub, g1_rW, g1_rU, g1_rb,
                           g1_hW, g1_hU, g1_hb,
                           att_qw, att_qb, att_kw, att_kb, att_vw, att_vb,
                           att_gate)
    hw0, s_out, s_in = _tc_node(x2, degp, wa0)
    aggp0 = _sc_aggregate(hw0, src, dst)
    hw1 = _tc_layer(aggp0, s_in, s_out, wa1)
    aggp1 = _sc_aggregate(hw1, src, dst)
    return _tc_final(aggp1, s_in)
